# baseline (speedup 1.0000x reference)
import jax
import jax.numpy as jnp
from jax import lax
from jax.experimental import pallas as pl
from jax.experimental.pallas import tpu as pltpu
from jax.experimental.pallas import tpu_sc as plsc

B = 16384
D = 64
NC = 2
NS = 16
NW = NC * NS
BPW = B // NW
CHUNK = 128
NCHUNK = BPW // CHUNK
LANES = 16


def _worker_base():
    wid = lax.axis_index("s") * NC + lax.axis_index("c")
    return wid, pl.multiple_of(wid * BPW, 128)


def _gather_body(idxb, table, bias, rows_out, bias_out, idx_v, rows_v, b_v, sem):
    wid, base = _worker_base()
    pltpu.sync_copy(idxb.at[pl.ds(wid * NCHUNK, NCHUNK)], idx_v)
    copies = []
    for j in range(NCHUNK):
        dst = pl.ds(j * CHUNK, CHUNK)
        copies.append(pltpu.async_copy(table.at[idx_v.at[j]], rows_v.at[dst], sem))
        copies.append(pltpu.async_copy(bias.at[idx_v.at[j]], b_v.at[dst], sem))
    for c in copies:
        c.wait()
    pltpu.sync_copy(rows_v, rows_out.at[pl.ds(base, BPW)])
    pltpu.sync_copy(b_v, bias_out.at[pl.ds(base, BPW)])


def _dot_body(u_rows, p_rows, partials, u_v, p_v, pacc, sem):
    wid, base = _worker_base()
    cu = pltpu.async_copy(u_rows.at[pl.ds(base, BPW)], u_v, sem)
    cp = pltpu.async_copy(p_rows.at[pl.ds(base, BPW)], p_v, sem)
    cu.wait()
    cp.wait()

    def dot_chunk(r, acc):
        s = acc
        for c in range(D // LANES):
            sl = pl.ds(c * LANES, LANES)
            s = s + u_v[r, sl] * p_v[r, sl]
        return s

    acc = lax.fori_loop(0, BPW, dot_chunk, jnp.zeros((LANES,), jnp.float32))
    pacc[...] = acc
    pltpu.sync_copy(pacc, partials.at[wid])


def _combine_body(partials, ubg, pbg, out, pall, bu_v, bp_v, ob):
    wid, base = _worker_base()
    pltpu.sync_copy(partials, pall)
    pltpu.sync_copy(ubg.at[pl.ds(base, BPW)], bu_v)
    pltpu.sync_copy(pbg.at[pl.ds(base, BPW)], bp_v)

    def sum_body(i, tv):
        return tv + pall[i, :]

    tv = lax.fori_loop(0, NW, sum_body, jnp.zeros((LANES,), jnp.float32))
    total = jnp.sum(tv)

    def sig_body(k, carry):
        sl = pl.ds(k * LANES, LANES)
        x = total + bu_v[sl] + bp_v[sl]
        ob[sl] = 1.0 / (1.0 + jnp.exp(-x))
        return carry

    lax.fori_loop(0, BPW // LANES, sig_body, 0)
    pltpu.sync_copy(ob, out.at[pl.ds(base, BPW)])


def kernel(inputs, user_emb, user_bias, place_emb, place_bias):
    u_idx = inputs[:, 0].astype(jnp.int32).reshape(B // CHUNK, CHUNK)
    p_idx = inputs[:, 1].astype(jnp.int32).reshape(B // CHUNK, CHUNK)
    ub = user_bias.reshape(-1)
    pb = place_bias.reshape(-1)

    def mesh():
        return plsc.VectorSubcoreMesh(core_axis_name="c", subcore_axis_name="s")

    gather_fn = pl.kernel(
        _gather_body,
        mesh=mesh(),
        compiler_params=pltpu.CompilerParams(use_tc_tiling_on_sc=False),
        out_type=(
            jax.ShapeDtypeStruct((B, D), jnp.float32),
            jax.ShapeDtypeStruct((B,), jnp.float32),
        ),
        scratch_types=[
            pltpu.VMEM((NCHUNK, CHUNK), jnp.int32),
            pltpu.VMEM((BPW, D), jnp.float32),
            pltpu.VMEM((BPW,), jnp.float32),
            pltpu.SemaphoreType.DMA,
        ],
    )
    u_rows, ub_g = gather_fn(u_idx, user_emb, ub)
    p_rows, pb_g = gather_fn(p_idx, place_emb, pb)

    dot_fn = pl.kernel(
        _dot_body,
        mesh=mesh(),
        compiler_params=pltpu.CompilerParams(use_tc_tiling_on_sc=False),
        out_type=jax.ShapeDtypeStruct((NW, LANES), jnp.float32),
        scratch_types=[
            pltpu.VMEM((BPW, D), jnp.float32),
            pltpu.VMEM((BPW, D), jnp.float32),
            pltpu.VMEM((LANES,), jnp.float32),
            pltpu.SemaphoreType.DMA,
        ],
    )
    partials = dot_fn(u_rows, p_rows)

    combine_fn = pl.kernel(
        _combine_body,
        mesh=mesh(),
        compiler_params=pltpu.CompilerParams(
            use_tc_tiling_on_sc=False, needs_layout_passes=False),
        out_type=jax.ShapeDtypeStruct((B,), jnp.float32),
        scratch_types=[
            pltpu.VMEM((NW, LANES), jnp.float32),
            pltpu.VMEM((BPW,), jnp.float32),
            pltpu.VMEM((BPW,), jnp.float32),
            pltpu.VMEM((BPW,), jnp.float32),
        ],
    )
    out = combine_fn(partials, ub_g, pb_g)
    return out.reshape(B, 1)


# speedup vs baseline: 3.3220x; 3.3220x over previous
import jax
import jax.numpy as jnp
from jax import lax
from jax.experimental import pallas as pl
from jax.experimental.pallas import tpu as pltpu
from jax.experimental.pallas import tpu_sc as plsc

B = 16384
D = 64
NC = 2
NS = 16
NW = NC * NS
BPW = B // NW
CHUNK = 128
NCHUNK = BPW // CHUNK
LANES = 16
NSLOT = 640
RING = 4


def _worker_base():
    wid = lax.axis_index("s") * NC + lax.axis_index("c")
    return wid, pl.multiple_of(wid * BPW, 128)


def _slab_plan(idx):
    order = jnp.argsort(idx).astype(jnp.int32)
    si = jnp.take(idx, order)
    sw = si.reshape(NW, BPW)
    slab = sw >> 7
    lane = sw & 127
    first = jnp.concatenate(
        [jnp.ones((NW, 1), bool), slab[:, 1:] != slab[:, :-1]], axis=1)
    slot = jnp.cumsum(first.astype(jnp.int32), axis=1) - 1
    nslab = (slot[:, -1] + 1).astype(jnp.int32)
    r_iota = lax.broadcasted_iota(jnp.int32, (NW, BPW), 1)
    w_iota = lax.broadcasted_iota(jnp.int32, (NW, BPW), 0)
    combo = jnp.broadcast_to(slab[:, -1:] * 1024 + BPW, (NW, NSLOT + 1))
    tgt = jnp.where(first, slot, NSLOT)
    combo = combo.at[w_iota, tgt].set(slab * 1024 + r_iota)
    combo = combo[:, :NSLOT].reshape(-1)
    return order, lane.reshape(-1), combo, nslab


def _sread(ref, i):
    return ref[pl.ds(i, LANES)][0]


def _slab_gather_body(tabT, combo_h, lane_h, dst_h, n_h, rows_out,
                      combo_v, lane_v, n_v, dst_v,
                      rows_v, ring0, ring1, ring2, ring3,
                      sem0, sem1, sem2, sem3, sem_m):
    wid, base = _worker_base()
    rings = (ring0, ring1, ring2, ring3)
    sems = (sem0, sem1, sem2, sem3)

    pltpu.sync_copy(combo_h.at[pl.ds(pl.multiple_of(wid * NSLOT, 128), NSLOT)],
                    combo_v.at[pl.ds(0, NSLOT)])
    pltpu.sync_copy(lane_h.at[pl.ds(base, BPW)], lane_v.at[pl.ds(0, BPW)])
    pltpu.sync_copy(n_h, n_v.at[pl.ds(0, NW)])
    for j in range(NCHUNK):
        pltpu.sync_copy(dst_h.at[pl.ds(base + j * CHUNK, CHUNK)], dst_v.at[j])
    n = _sread(n_v, wid)

    def issue(s, ring_k, sem_k):
        c = lax.shift_right_logical(_sread(combo_v, s), 10)
        cb = pl.multiple_of(c * 128, 128)
        pltpu.async_copy(tabT.at[:, pl.ds(cb, 128)], ring_k, sem_k)

    for k in range(RING):
        @pl.when(k < n)
        def _(k=k):
            issue(k, rings[k], sems[k])

    iotas = [lax.iota(jnp.int32, 16) + (16 * c4) for c4 in range(D // LANES)]

    def group(gi, carry):
        for k in range(RING):
            s = gi * RING + k
            ring_k, sem_k = rings[k], sems[k]

            @pl.when(s < n)
            def _(s=s, ring_k=ring_k, sem_k=sem_k):
                pltpu.make_async_copy(tabT.at[:, pl.ds(0, 128)],
                                      ring_k, sem_k).wait()
                rs = _sread(combo_v, s) & 1023
                re = _sread(combo_v, s + 1) & 1023

                def rowb(r, cc):
                    lv = jnp.full((16,), _sread(lane_v, r), jnp.int32)
                    for c4 in range(D // LANES):
                        g = plsc.load_gather(ring_k, [iotas[c4], lv])
                        rows_v[r, pl.ds(16 * c4, 16)] = g
                    return cc

                lax.fori_loop(rs, re, rowb, 0)

                @pl.when(s + RING < n)
                def _():
                    issue(s + RING, ring_k, sem_k)
        return carry

    lax.fori_loop(0, NSLOT // RING, group, 0)

    for j in range(NCHUNK):
        pltpu.async_copy(rows_v.at[pl.ds(j * CHUNK, CHUNK)],
                         rows_out.at[dst_v.at[j]], sem_m)
    for j in range(NCHUNK):
        pltpu.make_async_copy(rows_v.at[pl.ds(j * CHUNK, CHUNK)],
                              rows_out.at[dst_v.at[0]], sem_m).wait()


def _bias_body(uidxb, pidxb, ubias, pbias, ubg_out, pbg_out,
               idx_u, idx_p, bu_v, bp_v, sem):
    wid, base = _worker_base()
    pltpu.sync_copy(uidxb.at[pl.ds(wid * NCHUNK, NCHUNK)], idx_u)
    pltpu.sync_copy(pidxb.at[pl.ds(wid * NCHUNK, NCHUNK)], idx_p)
    copies = []
    for j in range(NCHUNK):
        dst = pl.ds(j * CHUNK, CHUNK)
        copies.append(pltpu.async_copy(ubias.at[idx_u.at[j]], bu_v.at[dst], sem))
        copies.append(pltpu.async_copy(pbias.at[idx_p.at[j]], bp_v.at[dst], sem))
    for c in copies:
        c.wait()
    pltpu.sync_copy(bu_v, ubg_out.at[pl.ds(base, BPW)])
    pltpu.sync_copy(bp_v, pbg_out.at[pl.ds(base, BPW)])


def _dot_body(u_rows, p_rows, partials, u_v, p_v, pacc, sem):
    wid, base = _worker_base()
    half = BPW // 2
    acc = jnp.zeros((LANES,), jnp.float32)
    for h in range(2):
        hb = pl.multiple_of(base + h * half, 128)
        cu = pltpu.async_copy(u_rows.at[pl.ds(hb, half)], u_v, sem)
        cp = pltpu.async_copy(p_rows.at[pl.ds(hb, half)], p_v, sem)
        cu.wait()
        cp.wait()

        def dot_chunk(r, a):
            s = a
            for c in range(D // LANES):
                sl = pl.ds(c * LANES, LANES)
                s = s + u_v[r, sl] * p_v[r, sl]
            return s

        acc = lax.fori_loop(0, half, dot_chunk, acc)
    for c in range(8):
        pacc[pl.ds(c * LANES, LANES)] = acc if c == 0 else jnp.zeros(
            (LANES,), jnp.float32)
    pltpu.sync_copy(pacc, partials.at[pl.ds(pl.multiple_of(wid * 128, 128), 128)])


def _combine_body(partials, ubg, pbg, out, pall, bu_v, bp_v, ob):
    wid, base = _worker_base()
    pltpu.sync_copy(partials, pall)
    pltpu.sync_copy(ubg.at[pl.ds(base, BPW)], bu_v)
    pltpu.sync_copy(pbg.at[pl.ds(base, BPW)], bp_v)

    def sum_body(i, tv):
        return tv + pall[pl.ds(i * 128, LANES)]

    tv = lax.fori_loop(0, NW, sum_body, jnp.zeros((LANES,), jnp.float32))
    total = jnp.sum(tv)

    def sig_body(k, carry):
        sl = pl.ds(k * LANES, LANES)
        x = total + bu_v[sl] + bp_v[sl]
        ob[sl] = 1.0 / (1.0 + jnp.exp(-x))
        return carry

    lax.fori_loop(0, BPW // LANES, sig_body, 0)
    pltpu.sync_copy(ob, out.at[pl.ds(base, BPW)])


def kernel(inputs, user_emb, user_bias, place_emb, place_bias):
    u_idx = inputs[:, 0].astype(jnp.int32)
    p_idx = inputs[:, 1].astype(jnp.int32)
    ub = user_bias.reshape(-1)
    pb = place_bias.reshape(-1)

    def mesh():
        return plsc.VectorSubcoreMesh(core_axis_name="c", subcore_axis_name="s")

    slab_fn = pl.kernel(
        _slab_gather_body,
        mesh=mesh(),
        compiler_params=pltpu.CompilerParams(needs_layout_passes=False),
        out_type=jax.ShapeDtypeStruct((B, 128), jnp.float32),
        scratch_types=[
            pltpu.VMEM((NSLOT + LANES,), jnp.int32),
            pltpu.VMEM((BPW + LANES,), jnp.int32),
            pltpu.VMEM((NW + LANES,), jnp.int32),
            pltpu.VMEM((NCHUNK, CHUNK), jnp.int32),
            pltpu.VMEM((BPW, 128), jnp.float32),
            pltpu.VMEM((D, 128), jnp.float32),
            pltpu.VMEM((D, 128), jnp.float32),
            pltpu.VMEM((D, 128), jnp.float32),
            pltpu.VMEM((D, 128), jnp.float32),
            pltpu.SemaphoreType.DMA,
            pltpu.SemaphoreType.DMA,
            pltpu.SemaphoreType.DMA,
            pltpu.SemaphoreType.DMA,
            pltpu.SemaphoreType.DMA,
        ],
    )
    order_u, lane_u, combo_u, n_u = _slab_plan(u_idx)
    order_p, lane_p, combo_p, n_p = _slab_plan(p_idx)
    u_rows = slab_fn(user_emb.T, combo_u, lane_u, order_u, n_u)
    p_rows = slab_fn(place_emb.T, combo_p, lane_p, order_p, n_p)

    bias_fn = pl.kernel(
        _bias_body,
        mesh=mesh(),
        compiler_params=pltpu.CompilerParams(use_tc_tiling_on_sc=False),
        out_type=(
            jax.ShapeDtypeStruct((B,), jnp.float32),
            jax.ShapeDtypeStruct((B,), jnp.float32),
        ),
        scratch_types=[
            pltpu.VMEM((NCHUNK, CHUNK), jnp.int32),
            pltpu.VMEM((NCHUNK, CHUNK), jnp.int32),
            pltpu.VMEM((BPW,), jnp.float32),
            pltpu.VMEM((BPW,), jnp.float32),
            pltpu.SemaphoreType.DMA,
        ],
    )
    ub_g, pb_g = bias_fn(u_idx.reshape(B // CHUNK, CHUNK),
                         p_idx.reshape(B // CHUNK, CHUNK), ub, pb)

    dot_fn = pl.kernel(
        _dot_body,
        mesh=mesh(),
        compiler_params=pltpu.CompilerParams(needs_layout_passes=False),
        out_type=jax.ShapeDtypeStruct((NW * 128,), jnp.float32),
        scratch_types=[
            pltpu.VMEM((BPW // 2, 128), jnp.float32),
            pltpu.VMEM((BPW // 2, 128), jnp.float32),
            pltpu.VMEM((128,), jnp.float32),
            pltpu.SemaphoreType.DMA,
        ],
    )
    partials = dot_fn(u_rows, p_rows)

    combine_fn = pl.kernel(
        _combine_body,
        mesh=mesh(),
        compiler_params=pltpu.CompilerParams(
            use_tc_tiling_on_sc=False, needs_layout_passes=False),
        out_type=jax.ShapeDtypeStruct((B,), jnp.float32),
        scratch_types=[
            pltpu.VMEM((NW * 128,), jnp.float32),
            pltpu.VMEM((BPW,), jnp.float32),
            pltpu.VMEM((BPW,), jnp.float32),
            pltpu.VMEM((BPW,), jnp.float32),
        ],
    )
    out = combine_fn(partials, ub_g, pb_g)
    return out.reshape(B, 1)


# speedup vs baseline: 3.5208x; 1.0598x over previous
import jax
import jax.numpy as jnp
from jax import lax
from jax.experimental import pallas as pl
from jax.experimental.pallas import tpu as pltpu
from jax.experimental.pallas import tpu_sc as plsc

B = 16384
D = 64
NC = 2
NS = 16
NW = NC * NS
BPW = B // NW
CHUNK = 128
NCHUNK = BPW // CHUNK
LANES = 16
NSLOT = 640
RING = 6


def _worker_base():
    wid = lax.axis_index("s") * NC + lax.axis_index("c")
    return wid, pl.multiple_of(wid * BPW, 128)


def _slab_plan(idx):
    order = jnp.argsort(idx, stable=False).astype(jnp.int32)
    si = jnp.take(idx, order)
    sw = si.reshape(NW, BPW)
    slab = sw >> 7
    lane = sw & 127
    first = jnp.concatenate(
        [jnp.ones((NW, 1), bool), slab[:, 1:] != slab[:, :-1]], axis=1)
    r_iota = lax.broadcasted_iota(jnp.int32, (NW, BPW), 1)
    pf = jnp.where(first, r_iota, BPW)
    row_start = jnp.sort(pf, axis=1)
    row_start = jnp.concatenate(
        [row_start, jnp.full((NW, NSLOT + 1 - BPW), BPW, jnp.int32)], axis=1)
    clamped = jnp.minimum(row_start, BPW - 1)
    slab_at = jnp.take_along_axis(slab, clamped, axis=1)
    combo = (slab_at * 1024 + row_start)[:, :NSLOT].reshape(-1)
    nslab = jnp.sum(first.astype(jnp.int32), axis=1).astype(jnp.int32)
    return order, lane.reshape(-1), combo, nslab


def _sread(ref, i):
    return ref[pl.ds(i, LANES)][0]


def _slab_gather_body(tabT, combo_h, lane_h, dst_h, n_h, rows_out,
                      combo_v, lane_v, n_v, dst_v,
                      rows_v, ring0, ring1, ring2, ring3, ring4, ring5,
                      sem0, sem1, sem2, sem3, sem4, sem5, sem_m):
    wid, base = _worker_base()
    rings = (ring0, ring1, ring2, ring3, ring4, ring5)
    sems = (sem0, sem1, sem2, sem3, sem4, sem5)

    pltpu.sync_copy(combo_h.at[pl.ds(pl.multiple_of(wid * NSLOT, 128), NSLOT)],
                    combo_v.at[pl.ds(0, NSLOT)])
    pltpu.sync_copy(lane_h.at[pl.ds(base, BPW)], lane_v.at[pl.ds(0, BPW)])
    pltpu.sync_copy(n_h, n_v.at[pl.ds(0, NW)])
    for j in range(NCHUNK):
        pltpu.sync_copy(dst_h.at[pl.ds(base + j * CHUNK, CHUNK)], dst_v.at[j])
    n = _sread(n_v, wid)

    def issue(s, ring_k, sem_k):
        c = lax.shift_right_logical(_sread(combo_v, s), 10)
        cb = pl.multiple_of(c * 128, 128)
        pltpu.async_copy(tabT.at[:, pl.ds(cb, 128)], ring_k, sem_k)

    for k in range(RING):
        @pl.when(k < n)
        def _(k=k):
            issue(k, rings[k], sems[k])

    iotas = [lax.iota(jnp.int32, 16) + (16 * c4) for c4 in range(D // LANES)]

    def group(gi, carry):
        for k in range(RING):
            s = gi * RING + k
            ring_k, sem_k = rings[k], sems[k]

            @pl.when(s < n)
            def _(s=s, ring_k=ring_k, sem_k=sem_k):
                pltpu.make_async_copy(tabT.at[:, pl.ds(0, 128)],
                                      ring_k, sem_k).wait()
                rs = _sread(combo_v, s) & 1023
                re = _sread(combo_v, s + 1) & 1023

                def rowb(r, cc):
                    lv = jnp.full((16,), _sread(lane_v, r), jnp.int32)
                    for c4 in range(D // LANES):
                        g = plsc.load_gather(ring_k, [iotas[c4], lv])
                        rows_v[r, pl.ds(16 * c4, 16)] = g
                    return cc

                lax.fori_loop(rs, re, rowb, 0)

                @pl.when(s + RING < n)
                def _():
                    issue(s + RING, ring_k, sem_k)
        return carry

    lax.fori_loop(0, NSLOT // RING, group, 0)

    for j in range(NCHUNK):
        pltpu.async_copy(rows_v.at[pl.ds(j * CHUNK, CHUNK)],
                         rows_out.at[dst_v.at[j]], sem_m)
    for j in range(NCHUNK):
        pltpu.make_async_copy(rows_v.at[pl.ds(j * CHUNK, CHUNK)],
                              rows_out.at[dst_v.at[0]], sem_m).wait()


def _bias_body(uidxb, pidxb, ubias, pbias, ubg_out, pbg_out,
               idx_u, idx_p, bu_v, bp_v, sem):
    wid, base = _worker_base()
    pltpu.sync_copy(uidxb.at[pl.ds(wid * NCHUNK, NCHUNK)], idx_u)
    pltpu.sync_copy(pidxb.at[pl.ds(wid * NCHUNK, NCHUNK)], idx_p)
    copies = []
    for j in range(NCHUNK):
        dst = pl.ds(j * CHUNK, CHUNK)
        copies.append(pltpu.async_copy(ubias.at[idx_u.at[j]], bu_v.at[dst], sem))
        copies.append(pltpu.async_copy(pbias.at[idx_p.at[j]], bp_v.at[dst], sem))
    for c in copies:
        c.wait()
    pltpu.sync_copy(bu_v, ubg_out.at[pl.ds(base, BPW)])
    pltpu.sync_copy(bp_v, pbg_out.at[pl.ds(base, BPW)])


def _dot_body(u_rows, p_rows, partials, u_v, p_v, pacc, sem):
    wid, base = _worker_base()
    half = BPW // 2
    acc = jnp.zeros((LANES,), jnp.float32)
    for h in range(2):
        hb = pl.multiple_of(base + h * half, 128)
        cu = pltpu.async_copy(u_rows.at[pl.ds(hb, half)], u_v, sem)
        cp = pltpu.async_copy(p_rows.at[pl.ds(hb, half)], p_v, sem)
        cu.wait()
        cp.wait()

        def dot_chunk(r, a):
            s = a
            for c in range(D // LANES):
                sl = pl.ds(c * LANES, LANES)
                s = s + u_v[r, sl] * p_v[r, sl]
            return s

        acc = lax.fori_loop(0, half, dot_chunk, acc)
    for c in range(8):
        pacc[pl.ds(c * LANES, LANES)] = acc if c == 0 else jnp.zeros(
            (LANES,), jnp.float32)
    pltpu.sync_copy(pacc, partials.at[pl.ds(pl.multiple_of(wid * 128, 128), 128)])


def _combine_body(partials, ubg, pbg, out, pall, bu_v, bp_v, ob):
    wid, base = _worker_base()
    pltpu.sync_copy(partials, pall)
    pltpu.sync_copy(ubg.at[pl.ds(base, BPW)], bu_v)
    pltpu.sync_copy(pbg.at[pl.ds(base, BPW)], bp_v)

    def sum_body(i, tv):
        return tv + pall[pl.ds(i * 128, LANES)]

    tv = lax.fori_loop(0, NW, sum_body, jnp.zeros((LANES,), jnp.float32))
    total = jnp.sum(tv)

    def sig_body(k, carry):
        sl = pl.ds(k * LANES, LANES)
        x = total + bu_v[sl] + bp_v[sl]
        ob[sl] = 1.0 / (1.0 + jnp.exp(-x))
        return carry

    lax.fori_loop(0, BPW // LANES, sig_body, 0)
    pltpu.sync_copy(ob, out.at[pl.ds(base, BPW)])


def kernel(inputs, user_emb, user_bias, place_emb, place_bias):
    u_idx = inputs[:, 0].astype(jnp.int32)
    p_idx = inputs[:, 1].astype(jnp.int32)
    ub = user_bias.reshape(-1)
    pb = place_bias.reshape(-1)

    def mesh():
        return plsc.VectorSubcoreMesh(core_axis_name="c", subcore_axis_name="s")

    slab_fn = pl.kernel(
        _slab_gather_body,
        mesh=mesh(),
        compiler_params=pltpu.CompilerParams(needs_layout_passes=False),
        out_type=jax.ShapeDtypeStruct((B, 128), jnp.float32),
        scratch_types=[
            pltpu.VMEM((NSLOT + LANES,), jnp.int32),
            pltpu.VMEM((BPW + LANES,), jnp.int32),
            pltpu.VMEM((NW + LANES,), jnp.int32),
            pltpu.VMEM((NCHUNK, CHUNK), jnp.int32),
            pltpu.VMEM((BPW, 128), jnp.float32),
            pltpu.VMEM((D, 128), jnp.float32),
            pltpu.VMEM((D, 128), jnp.float32),
            pltpu.VMEM((D, 128), jnp.float32),
            pltpu.VMEM((D, 128), jnp.float32),
            pltpu.VMEM((D, 128), jnp.float32),
            pltpu.VMEM((D, 128), jnp.float32),
            pltpu.SemaphoreType.DMA,
            pltpu.SemaphoreType.DMA,
            pltpu.SemaphoreType.DMA,
            pltpu.SemaphoreType.DMA,
            pltpu.SemaphoreType.DMA,
            pltpu.SemaphoreType.DMA,
            pltpu.SemaphoreType.DMA,
        ],
    )
    order_u, lane_u, combo_u, n_u = _slab_plan(u_idx)
    order_p, lane_p, combo_p, n_p = _slab_plan(p_idx)
    u_rows = slab_fn(user_emb.T, combo_u, lane_u, order_u, n_u)
    p_rows = slab_fn(place_emb.T, combo_p, lane_p, order_p, n_p)

    bias_fn = pl.kernel(
        _bias_body,
        mesh=mesh(),
        compiler_params=pltpu.CompilerParams(use_tc_tiling_on_sc=False),
        out_type=(
            jax.ShapeDtypeStruct((B,), jnp.float32),
            jax.ShapeDtypeStruct((B,), jnp.float32),
        ),
        scratch_types=[
            pltpu.VMEM((NCHUNK, CHUNK), jnp.int32),
            pltpu.VMEM((NCHUNK, CHUNK), jnp.int32),
            pltpu.VMEM((BPW,), jnp.float32),
            pltpu.VMEM((BPW,), jnp.float32),
            pltpu.SemaphoreType.DMA,
        ],
    )
    ub_g, pb_g = bias_fn(u_idx.reshape(B // CHUNK, CHUNK),
                         p_idx.reshape(B // CHUNK, CHUNK), ub, pb)

    dot_fn = pl.kernel(
        _dot_body,
        mesh=mesh(),
        compiler_params=pltpu.CompilerParams(needs_layout_passes=False),
        out_type=jax.ShapeDtypeStruct((NW * 128,), jnp.float32),
        scratch_types=[
            pltpu.VMEM((BPW // 2, 128), jnp.float32),
            pltpu.VMEM((BPW // 2, 128), jnp.float32),
            pltpu.VMEM((128,), jnp.float32),
            pltpu.SemaphoreType.DMA,
        ],
    )
    partials = dot_fn(u_rows, p_rows)

    combine_fn = pl.kernel(
        _combine_body,
        mesh=mesh(),
        compiler_params=pltpu.CompilerParams(
            use_tc_tiling_on_sc=False, needs_layout_passes=False),
        out_type=jax.ShapeDtypeStruct((B,), jnp.float32),
        scratch_types=[
            pltpu.VMEM((NW * 128,), jnp.float32),
            pltpu.VMEM((BPW,), jnp.float32),
            pltpu.VMEM((BPW,), jnp.float32),
            pltpu.VMEM((BPW,), jnp.float32),
        ],
    )
    out = combine_fn(partials, ub_g, pb_g)
    return out.reshape(B, 1)


# speedup vs baseline: 3.5250x; 1.0012x over previous
import jax
import jax.numpy as jnp
from jax import lax
from jax.experimental import pallas as pl
from jax.experimental.pallas import tpu as pltpu
from jax.experimental.pallas import tpu_sc as plsc

B = 16384
D = 64
NC = 2
NS = 16
NW = NC * NS
BPW = B // NW
CHUNK = 128
NCHUNK = BPW // CHUNK
LANES = 16
NSLOT = 640
RING = 6


def _worker_base():
    wid = lax.axis_index("s") * NC + lax.axis_index("c")
    return wid, pl.multiple_of(wid * BPW, 128)


def _slab_plan(idx):
    order = jnp.argsort(idx, stable=False).astype(jnp.int32)
    si = jnp.take(idx, order)
    sw = si.reshape(NW, BPW)
    slab = sw >> 7
    first = jnp.concatenate(
        [jnp.ones((NW, 1), bool), slab[:, 1:] != slab[:, :-1]], axis=1)
    r_iota = lax.broadcasted_iota(jnp.int32, (NW, BPW), 1)
    pf = jnp.where(first, r_iota, BPW)
    row_start = jnp.sort(pf, axis=1)
    row_start = jnp.concatenate(
        [row_start, jnp.full((NW, NSLOT - BPW), BPW, jnp.int32)], axis=1)
    nslab = jnp.sum(first.astype(jnp.int32), axis=1).astype(jnp.int32)
    return order, si, row_start.reshape(-1), nslab


def _sread(ref, i):
    return ref[pl.ds(i, LANES)][0]


def _slab_gather_body(tabT, combo_h, si_h, dst_h, n_h, rows_out,
                      combo_v, si_v, n_v, dst_v,
                      rows_v, ring0, ring1, ring2, ring3, ring4, ring5,
                      sem0, sem1, sem2, sem3, sem4, sem5, sem_m):
    wid, base = _worker_base()
    rings = (ring0, ring1, ring2, ring3, ring4, ring5)
    sems = (sem0, sem1, sem2, sem3, sem4, sem5)

    pltpu.sync_copy(combo_h.at[pl.ds(pl.multiple_of(wid * NSLOT, 128), NSLOT)],
                    combo_v.at[pl.ds(0, NSLOT)])
    pltpu.sync_copy(si_h.at[pl.ds(base, BPW)], si_v.at[pl.ds(0, BPW)])
    pltpu.sync_copy(n_h, n_v.at[pl.ds(0, NW)])
    for j in range(NCHUNK):
        pltpu.sync_copy(dst_h.at[pl.ds(base + j * CHUNK, CHUNK)], dst_v.at[j])
    n = _sread(n_v, wid)

    def issue(s, ring_k, sem_k):
        c = lax.shift_right_logical(_sread(si_v, _sread(combo_v, s)), 7)
        cb = pl.multiple_of(c * 128, 128)
        pltpu.async_copy(tabT.at[:, pl.ds(cb, 128)], ring_k, sem_k)

    for k in range(RING):
        @pl.when(k < n)
        def _(k=k):
            issue(k, rings[k], sems[k])

    iotas = [lax.iota(jnp.int32, 16) + (16 * c4) for c4 in range(D // LANES)]

    def group(gi, carry):
        for k in range(RING):
            s = gi * RING + k
            ring_k, sem_k = rings[k], sems[k]

            @pl.when(s < n)
            def _(s=s, ring_k=ring_k, sem_k=sem_k):
                pltpu.make_async_copy(tabT.at[:, pl.ds(0, 128)],
                                      ring_k, sem_k).wait()
                rs = _sread(combo_v, s)
                re = _sread(combo_v, s + 1)

                def rowb(r, cc):
                    lv = jnp.full((16,), _sread(si_v, r) & 127, jnp.int32)
                    for c4 in range(D // LANES):
                        g = plsc.load_gather(ring_k, [iotas[c4], lv])
                        rows_v[r, pl.ds(16 * c4, 16)] = g
                    return cc

                lax.fori_loop(rs, re, rowb, 0)

                @pl.when(s + RING < n)
                def _():
                    issue(s + RING, ring_k, sem_k)
        return carry

    lax.fori_loop(0, NSLOT // RING, group, 0)

    for j in range(NCHUNK):
        pltpu.async_copy(rows_v.at[pl.ds(j * CHUNK, CHUNK)],
                         rows_out.at[dst_v.at[j]], sem_m)
    for j in range(NCHUNK):
        pltpu.make_async_copy(rows_v.at[pl.ds(j * CHUNK, CHUNK)],
                              rows_out.at[dst_v.at[0]], sem_m).wait()


def _dot_body(u_rows, p_rows, partials, u_v, p_v, pacc, sem):
    wid, base = _worker_base()
    half = BPW // 2
    acc = jnp.zeros((LANES,), jnp.float32)
    for h in range(2):
        hb = pl.multiple_of(base + h * half, 128)
        cu = pltpu.async_copy(u_rows.at[pl.ds(hb, half)], u_v, sem)
        cp = pltpu.async_copy(p_rows.at[pl.ds(hb, half)], p_v, sem)
        cu.wait()
        cp.wait()

        def dot_chunk(r, a):
            s = a
            for c in range(D // LANES):
                sl = pl.ds(c * LANES, LANES)
                s = s + u_v[r, sl] * p_v[r, sl]
            return s

        acc = lax.fori_loop(0, half, dot_chunk, acc)
    for c in range(8):
        pacc[pl.ds(c * LANES, LANES)] = acc if c == 0 else jnp.zeros(
            (LANES,), jnp.float32)
    pltpu.sync_copy(pacc, partials.at[pl.ds(pl.multiple_of(wid * 128, 128), 128)])


def _combine_body(partials, uidxb, pidxb, ubias, pbias, out,
                  pall, idx_u, idx_p, bu_v, bp_v, ob, sem):
    wid, base = _worker_base()
    pltpu.sync_copy(uidxb.at[pl.ds(wid * NCHUNK, NCHUNK)], idx_u)
    pltpu.sync_copy(pidxb.at[pl.ds(wid * NCHUNK, NCHUNK)], idx_p)
    copies = []
    for j in range(NCHUNK):
        dst = pl.ds(j * CHUNK, CHUNK)
        copies.append(pltpu.async_copy(ubias.at[idx_u.at[j]], bu_v.at[dst], sem))
        copies.append(pltpu.async_copy(pbias.at[idx_p.at[j]], bp_v.at[dst], sem))
    pltpu.sync_copy(partials, pall)
    for c in copies:
        c.wait()

    def sum_body(i, tv):
        return tv + pall[pl.ds(i * 128, LANES)]

    tv = lax.fori_loop(0, NW, sum_body, jnp.zeros((LANES,), jnp.float32))
    total = jnp.sum(tv)

    def sig_body(k, carry):
        sl = pl.ds(k * LANES, LANES)
        x = total + bu_v[sl] + bp_v[sl]
        ob[sl] = 1.0 / (1.0 + jnp.exp(-x))
        return carry

    lax.fori_loop(0, BPW // LANES, sig_body, 0)
    pltpu.sync_copy(ob, out.at[pl.ds(base, BPW)])


def kernel(inputs, user_emb, user_bias, place_emb, place_bias):
    u_idx = inputs[:, 0].astype(jnp.int32)
    p_idx = inputs[:, 1].astype(jnp.int32)
    ub = user_bias.reshape(-1)
    pb = place_bias.reshape(-1)

    def mesh():
        return plsc.VectorSubcoreMesh(core_axis_name="c", subcore_axis_name="s")

    slab_fn = pl.kernel(
        _slab_gather_body,
        mesh=mesh(),
        compiler_params=pltpu.CompilerParams(needs_layout_passes=False),
        out_type=jax.ShapeDtypeStruct((B, 128), jnp.float32),
        scratch_types=[
            pltpu.VMEM((NSLOT + LANES,), jnp.int32),
            pltpu.VMEM((BPW + LANES,), jnp.int32),
            pltpu.VMEM((NW + LANES,), jnp.int32),
            pltpu.VMEM((NCHUNK, CHUNK), jnp.int32),
            pltpu.VMEM((BPW, 128), jnp.float32),
            pltpu.VMEM((D, 128), jnp.float32),
            pltpu.VMEM((D, 128), jnp.float32),
            pltpu.VMEM((D, 128), jnp.float32),
            pltpu.VMEM((D, 128), jnp.float32),
            pltpu.VMEM((D, 128), jnp.float32),
            pltpu.VMEM((D, 128), jnp.float32),
            pltpu.SemaphoreType.DMA,
            pltpu.SemaphoreType.DMA,
            pltpu.SemaphoreType.DMA,
            pltpu.SemaphoreType.DMA,
            pltpu.SemaphoreType.DMA,
            pltpu.SemaphoreType.DMA,
            pltpu.SemaphoreType.DMA,
        ],
    )
    order_u, si_u, combo_u, n_u = _slab_plan(u_idx)
    order_p, si_p, combo_p, n_p = _slab_plan(p_idx)
    u_rows = slab_fn(user_emb.T, combo_u, si_u, order_u, n_u)
    p_rows = slab_fn(place_emb.T, combo_p, si_p, order_p, n_p)

    dot_fn = pl.kernel(
        _dot_body,
        mesh=mesh(),
        compiler_params=pltpu.CompilerParams(needs_layout_passes=False),
        out_type=jax.ShapeDtypeStruct((NW * 128,), jnp.float32),
        scratch_types=[
            pltpu.VMEM((BPW // 2, 128), jnp.float32),
            pltpu.VMEM((BPW // 2, 128), jnp.float32),
            pltpu.VMEM((128,), jnp.float32),
            pltpu.SemaphoreType.DMA,
        ],
    )
    partials = dot_fn(u_rows, p_rows)

    combine_fn = pl.kernel(
        _combine_body,
        mesh=mesh(),
        compiler_params=pltpu.CompilerParams(
            use_tc_tiling_on_sc=False, needs_layout_passes=False),
        out_type=jax.ShapeDtypeStruct((B,), jnp.float32),
        scratch_types=[
            pltpu.VMEM((NW * 128,), jnp.float32),
            pltpu.VMEM((NCHUNK, CHUNK), jnp.int32),
            pltpu.VMEM((NCHUNK, CHUNK), jnp.int32),
            pltpu.VMEM((BPW,), jnp.float32),
            pltpu.VMEM((BPW,), jnp.float32),
            pltpu.VMEM((BPW,), jnp.float32),
            pltpu.SemaphoreType.DMA,
        ],
    )
    out = combine_fn(partials, u_idx.reshape(B // CHUNK, CHUNK),
                     p_idx.reshape(B // CHUNK, CHUNK), ub, pb)
    return out.reshape(B, 1)


# speedup vs baseline: 3.5250x; 1.0000x over previous
import jax
import jax.numpy as jnp
from jax import lax
from jax.experimental import pallas as pl
from jax.experimental.pallas import tpu as pltpu
from jax.experimental.pallas import tpu_sc as plsc

B = 16384
D = 64
NC = 2
NS = 16
NW = NC * NS
BPW = B // NW
CHUNK = 128
NCHUNK = BPW // CHUNK
LANES = 16
NSLOT = 640
RING = 6


def _worker_base():
    wid = lax.axis_index("s") * NC + lax.axis_index("c")
    return wid, pl.multiple_of(wid * BPW, 128)


def _slab_plan(idx):
    order = jnp.argsort((idx >> 7).astype(jnp.uint16),
                        stable=False).astype(jnp.int32)
    si = jnp.take(idx, order)
    sw = si.reshape(NW, BPW)
    slab = sw >> 7
    first = jnp.concatenate(
        [jnp.ones((NW, 1), bool), slab[:, 1:] != slab[:, :-1]], axis=1)
    r_iota = lax.broadcasted_iota(jnp.int32, (NW, BPW), 1)
    pf = jnp.where(first, r_iota, BPW)
    row_start = jnp.sort(pf, axis=1)
    row_start = jnp.concatenate(
        [row_start, jnp.full((NW, NSLOT - BPW), BPW, jnp.int32)], axis=1)
    nslab = jnp.sum(first.astype(jnp.int32), axis=1).astype(jnp.int32)
    return order, si, row_start.reshape(-1), nslab


def _sread(ref, i):
    return ref[pl.ds(i, LANES)][0]


def _slab_gather_body(tabT, combo_h, si_h, dst_h, n_h, rows_out,
                      combo_v, si_v, n_v, dst_v,
                      rows_v, ring0, ring1, ring2, ring3, ring4, ring5,
                      sem0, sem1, sem2, sem3, sem4, sem5, sem_m):
    wid, base = _worker_base()
    rings = (ring0, ring1, ring2, ring3, ring4, ring5)
    sems = (sem0, sem1, sem2, sem3, sem4, sem5)

    pltpu.sync_copy(combo_h.at[pl.ds(pl.multiple_of(wid * NSLOT, 128), NSLOT)],
                    combo_v.at[pl.ds(0, NSLOT)])
    pltpu.sync_copy(si_h.at[pl.ds(base, BPW)], si_v.at[pl.ds(0, BPW)])
    pltpu.sync_copy(n_h, n_v.at[pl.ds(0, NW)])
    for j in range(NCHUNK):
        pltpu.sync_copy(dst_h.at[pl.ds(base + j * CHUNK, CHUNK)], dst_v.at[j])
    n = _sread(n_v, wid)

    def issue(s, ring_k, sem_k):
        c = lax.shift_right_logical(_sread(si_v, _sread(combo_v, s)), 7)
        cb = pl.multiple_of(c * 128, 128)
        pltpu.async_copy(tabT.at[:, pl.ds(cb, 128)], ring_k, sem_k)

    for k in range(RING):
        @pl.when(k < n)
        def _(k=k):
            issue(k, rings[k], sems[k])

    iotas = [lax.iota(jnp.int32, 16) + (16 * c4) for c4 in range(D // LANES)]

    def group(gi, carry):
        for k in range(RING):
            s = gi * RING + k
            ring_k, sem_k = rings[k], sems[k]

            @pl.when(s < n)
            def _(s=s, ring_k=ring_k, sem_k=sem_k):
                pltpu.make_async_copy(tabT.at[:, pl.ds(0, 128)],
                                      ring_k, sem_k).wait()
                rs = _sread(combo_v, s)
                re = _sread(combo_v, s + 1)

                def rowb(r, cc):
                    lv = jnp.full((16,), _sread(si_v, r) & 127, jnp.int32)
                    for c4 in range(D // LANES):
                        g = plsc.load_gather(ring_k, [iotas[c4], lv])
                        rows_v[r, pl.ds(16 * c4, 16)] = g
                    return cc

                lax.fori_loop(rs, re, rowb, 0)

                @pl.when(s + RING < n)
                def _():
                    issue(s + RING, ring_k, sem_k)
        return carry

    lax.fori_loop(0, NSLOT // RING, group, 0)

    for j in range(NCHUNK):
        pltpu.async_copy(rows_v.at[pl.ds(j * CHUNK, CHUNK)],
                         rows_out.at[dst_v.at[j]], sem_m)
    for j in range(NCHUNK):
        pltpu.make_async_copy(rows_v.at[pl.ds(j * CHUNK, CHUNK)],
                              rows_out.at[dst_v.at[0]], sem_m).wait()


def _dot_body(u_rows, p_rows, partials, u_v, p_v, pacc, sem):
    wid, base = _worker_base()
    half = BPW // 2
    acc = jnp.zeros((LANES,), jnp.float32)
    for h in range(2):
        hb = pl.multiple_of(base + h * half, 128)
        cu = pltpu.async_copy(u_rows.at[pl.ds(hb, half)], u_v, sem)
        cp = pltpu.async_copy(p_rows.at[pl.ds(hb, half)], p_v, sem)
        cu.wait()
        cp.wait()

        def dot_chunk(r, a):
            s = a
            for c in range(D // LANES):
                sl = pl.ds(c * LANES, LANES)
                s = s + u_v[r, sl] * p_v[r, sl]
            return s

        acc = lax.fori_loop(0, half, dot_chunk, acc)
    for c in range(8):
        pacc[pl.ds(c * LANES, LANES)] = acc if c == 0 else jnp.zeros(
            (LANES,), jnp.float32)
    pltpu.sync_copy(pacc, partials.at[pl.ds(pl.multiple_of(wid * 128, 128), 128)])


def _combine_body(partials, uidxb, pidxb, ubias, pbias, out,
                  pall, idx_u, idx_p, bu_v, bp_v, ob, sem):
    wid, base = _worker_base()
    pltpu.sync_copy(uidxb.at[pl.ds(wid * NCHUNK, NCHUNK)], idx_u)
    pltpu.sync_copy(pidxb.at[pl.ds(wid * NCHUNK, NCHUNK)], idx_p)
    copies = []
    for j in range(NCHUNK):
        dst = pl.ds(j * CHUNK, CHUNK)
        copies.append(pltpu.async_copy(ubias.at[idx_u.at[j]], bu_v.at[dst], sem))
        copies.append(pltpu.async_copy(pbias.at[idx_p.at[j]], bp_v.at[dst], sem))
    pltpu.sync_copy(partials, pall)
    for c in copies:
        c.wait()

    def sum_body(i, tv):
        return tv + pall[pl.ds(i * 128, LANES)]

    tv = lax.fori_loop(0, NW, sum_body, jnp.zeros((LANES,), jnp.float32))
    total = jnp.sum(tv)

    def sig_body(k, carry):
        sl = pl.ds(k * LANES, LANES)
        x = total + bu_v[sl] + bp_v[sl]
        ob[sl] = 1.0 / (1.0 + jnp.exp(-x))
        return carry

    lax.fori_loop(0, BPW // LANES, sig_body, 0)
    pltpu.sync_copy(ob, out.at[pl.ds(base, BPW)])


def kernel(inputs, user_emb, user_bias, place_emb, place_bias):
    u_idx = inputs[:, 0].astype(jnp.int32)
    p_idx = inputs[:, 1].astype(jnp.int32)
    ub = user_bias.reshape(-1)
    pb = place_bias.reshape(-1)

    def mesh():
        return plsc.VectorSubcoreMesh(core_axis_name="c", subcore_axis_name="s")

    slab_fn = pl.kernel(
        _slab_gather_body,
        mesh=mesh(),
        compiler_params=pltpu.CompilerParams(needs_layout_passes=False),
        out_type=jax.ShapeDtypeStruct((B, 128), jnp.float32),
        scratch_types=[
            pltpu.VMEM((NSLOT + LANES,), jnp.int32),
            pltpu.VMEM((BPW + LANES,), jnp.int32),
            pltpu.VMEM((NW + LANES,), jnp.int32),
            pltpu.VMEM((NCHUNK, CHUNK), jnp.int32),
            pltpu.VMEM((BPW, 128), jnp.float32),
            pltpu.VMEM((D, 128), jnp.float32),
            pltpu.VMEM((D, 128), jnp.float32),
            pltpu.VMEM((D, 128), jnp.float32),
            pltpu.VMEM((D, 128), jnp.float32),
            pltpu.VMEM((D, 128), jnp.float32),
            pltpu.VMEM((D, 128), jnp.float32),
            pltpu.SemaphoreType.DMA,
            pltpu.SemaphoreType.DMA,
            pltpu.SemaphoreType.DMA,
            pltpu.SemaphoreType.DMA,
            pltpu.SemaphoreType.DMA,
            pltpu.SemaphoreType.DMA,
            pltpu.SemaphoreType.DMA,
        ],
    )
    order_u, si_u, combo_u, n_u = _slab_plan(u_idx)
    order_p, si_p, combo_p, n_p = _slab_plan(p_idx)
    u_rows = slab_fn(user_emb.T, combo_u, si_u, order_u, n_u)
    p_rows = slab_fn(place_emb.T, combo_p, si_p, order_p, n_p)

    dot_fn = pl.kernel(
        _dot_body,
        mesh=mesh(),
        compiler_params=pltpu.CompilerParams(needs_layout_passes=False),
        out_type=jax.ShapeDtypeStruct((NW * 128,), jnp.float32),
        scratch_types=[
            pltpu.VMEM((BPW // 2, 128), jnp.float32),
            pltpu.VMEM((BPW // 2, 128), jnp.float32),
            pltpu.VMEM((128,), jnp.float32),
            pltpu.SemaphoreType.DMA,
        ],
    )
    partials = dot_fn(u_rows, p_rows)

    combine_fn = pl.kernel(
        _combine_body,
        mesh=mesh(),
        compiler_params=pltpu.CompilerParams(
            use_tc_tiling_on_sc=False, needs_layout_passes=False),
        out_type=jax.ShapeDtypeStruct((B,), jnp.float32),
        scratch_types=[
            pltpu.VMEM((NW * 128,), jnp.float32),
            pltpu.VMEM((NCHUNK, CHUNK), jnp.int32),
            pltpu.VMEM((NCHUNK, CHUNK), jnp.int32),
            pltpu.VMEM((BPW,), jnp.float32),
            pltpu.VMEM((BPW,), jnp.float32),
            pltpu.VMEM((BPW,), jnp.float32),
            pltpu.SemaphoreType.DMA,
        ],
    )
    out = combine_fn(partials, u_idx.reshape(B // CHUNK, CHUNK),
                     p_idx.reshape(B // CHUNK, CHUNK), ub, pb)
    return out.reshape(B, 1)
